# Initial kernel scaffold; baseline (speedup 1.0000x reference)
import jax
import jax.numpy as jnp
from jax.experimental import pallas as pl


def kernel(z, W_ds, b_ds, W_as, b_as, W_dn, b_dn, dataset_id, assay_id, donor_id):
    raise NotImplementedError("write your pallas kernel here")


# speedup vs baseline: 4.2330x; 4.2330x over previous
import functools

import jax
import jax.numpy as jnp
from jax import lax
from jax.experimental import pallas as pl

LATENT = 128
N_TOKENS = 4096
BLK = 128
GRID = N_TOKENS // BLK


def _stage(y, wt_ref, b_ref, id_ref, n_experts):
    ids = id_ref[0]
    h = jnp.dot(y, wt_ref[...], preferred_element_type=jnp.float32)
    h3 = h.reshape(BLK, n_experts, LATENT)
    onehot = (ids == lax.broadcasted_iota(jnp.int32, (BLK, n_experts), 1))
    onehot = onehot.astype(jnp.float32)
    out = jnp.sum(h3 * onehot[:, :, None], axis=1)
    out = out + jnp.dot(onehot, b_ref[...], preferred_element_type=jnp.float32)
    return jnp.maximum(out, 0.0)


def _body(z_ref, wds_ref, bds_ref, was_ref, bas_ref, wdn_ref, bdn_ref,
          ids_ds_ref, ids_as_ref, ids_dn_ref, out_ref):
    y = z_ref[...]
    y = _stage(y, wds_ref, bds_ref, ids_ds_ref, 64)
    y = _stage(y, was_ref, bas_ref, ids_as_ref, 16)
    y = _stage(y, wdn_ref, bdn_ref, ids_dn_ref, 8)
    out_ref[...] = y


@jax.jit
def kernel(z, W_ds, b_ds, W_as, b_as, W_dn, b_dn, dataset_id, assay_id, donor_id):
    d = LATENT
    wt_ds = W_ds.transpose(1, 0, 2).reshape(d, -1)
    wt_as = W_as.transpose(1, 0, 2).reshape(d, -1)
    wt_dn = W_dn.transpose(1, 0, 2).reshape(d, -1)
    ids_ds = dataset_id.astype(jnp.int32).reshape(GRID, BLK, 1)
    ids_as = assay_id.astype(jnp.int32).reshape(GRID, BLK, 1)
    ids_dn = donor_id.astype(jnp.int32).reshape(GRID, BLK, 1)

    full = lambda shape: pl.BlockSpec(shape, lambda i: (0,) * len(shape))
    out = pl.pallas_call(
        _body,
        grid=(GRID,),
        in_specs=[
            pl.BlockSpec((BLK, d), lambda i: (i, 0)),
            full(wt_ds.shape), full(b_ds.shape),
            full(wt_as.shape), full(b_as.shape),
            full(wt_dn.shape), full(b_dn.shape),
            pl.BlockSpec((1, BLK, 1), lambda i: (i, 0, 0)),
            pl.BlockSpec((1, BLK, 1), lambda i: (i, 0, 0)),
            pl.BlockSpec((1, BLK, 1), lambda i: (i, 0, 0)),
        ],
        out_specs=pl.BlockSpec((BLK, d), lambda i: (i, 0)),
        out_shape=jax.ShapeDtypeStruct((N_TOKENS, d), jnp.float32),
    )(z, wt_ds, b_ds, wt_as, b_as, wt_dn, b_dn, ids_ds, ids_as, ids_dn)
    return out
